# baseline (speedup 1.0000x reference)
import functools

import jax
import jax.numpy as jnp
from jax import lax
from jax.experimental import pallas as pl
from jax.experimental.pallas import tpu as pltpu
from jax.experimental.pallas import tpu_sc as plsc

NUM_CORES = 2
NUM_SUBCORES = 16
NW = NUM_CORES * NUM_SUBCORES

T_ROWS = 16384
J_COLS = 26
EMB = 64
N_VOCAB = 1_000_000
N_PAIRS = N_VOCAB // 2

TPW = T_ROWS // NW
CPW = TPW // 128
UNITS = J_COLS * CPW

PACK_COLS = 512
UNPACK_T = 4096
TAIL_START = (N_VOCAB // PACK_COLS) * PACK_COLS
TAIL_P0 = TAIL_START // 2


def _eye(n):
  return (
      lax.broadcasted_iota(jnp.int32, (n, n), 0)
      == lax.broadcasted_iota(jnp.int32, (n, n), 1)
  ).astype(jnp.float32)


PACK_SUB = 63
PACK_BLK = PACK_COLS * PACK_SUB


def _tc_pack_body(wt_ref, out_ref):
  x = wt_ref[...]
  z = lax.dot_general(x, _eye(EMB), (((0,), (0,)), ((), ())))
  for s in range(PACK_SUB):
    out_ref[pl.ds(s * 256, 256), 0:EMB] = z[s * 512:s * 512 + 256]
    out_ref[pl.ds(s * 256, 256), EMB:128] = z[s * 512 + 256:s * 512 + 512]


def _tc_pack(wt):
  grid = N_VOCAB // PACK_BLK
  return pl.pallas_call(
      _tc_pack_body,
      grid=(grid,),
      in_specs=[pl.BlockSpec((EMB, PACK_BLK), lambda k: (0, k))],
      out_specs=pl.BlockSpec((PACK_BLK // 2, 128), lambda k: (k, 0)),
      out_shape=jax.ShapeDtypeStruct((N_PAIRS, 128), jnp.float32),
  )(wt)


def _make_sc_gather(tpw):
  t_rows = tpw * NW
  cpw = tpw // 128
  units = J_COLS * cpw

  @functools.partial(
      pl.kernel,
      out_type=jax.ShapeDtypeStruct((J_COLS, t_rows, 128), jnp.float32),
      mesh=plsc.VectorSubcoreMesh(
          core_axis_name="c",
          subcore_axis_name="s",
          num_cores=NUM_CORES,
          num_subcores=NUM_SUBCORES,
      ),
      scratch_types=[
          pltpu.VMEM((J_COLS, tpw), jnp.int32),
          pltpu.VMEM((J_COLS * tpw,), jnp.int32),
          pltpu.VMEM((128, 128), jnp.float32),
          pltpu.VMEM((128, 128), jnp.float32),
          pltpu.SemaphoreType.DMA,
          pltpu.SemaphoreType.DMA,
      ],
      compiler_params=pltpu.CompilerParams(use_tc_tiling_on_sc=True),
  )
  def sc_gather(tok_hbm, w_hbm, out_hbm, tok_v, idxp, pbuf0, pbuf1,
                sem0, sem1):
    c = lax.axis_index("c")
    s = lax.axis_index("s")
    wid = s * NUM_CORES + c
    sems = (sem0, sem1)
    bufs = (pbuf0, pbuf1)

    pltpu.sync_copy(tok_hbm.at[:, wid], tok_v)

    for j in range(J_COLS):
      def build(g, carry, j=j):
        v = tok_v[j, pl.ds(g * 16, 16)]
        p = lax.shift_left(
            lax.shift_right_logical(v, 9), 8
        ) + lax.bitwise_and(v, 255)
        p_tail = TAIL_P0 + lax.shift_right_logical(v - TAIL_START, 1)
        p = lax.select(v >= TAIL_START, p_tail, p)
        idxp[pl.ds(j * tpw + g * 16, 16)] = p
        return carry
      lax.fori_loop(0, tpw // 16, build, 0)

    def fire(u, b):
      pltpu.async_copy(
          w_hbm.at[idxp.at[pl.ds(u * 128, 128)]], bufs[b], sems[b])

    def proc(u, b):
      pltpu.make_async_copy(
          w_hbm.at[idxp.at[pl.ds(u * 128, 128)]], bufs[b], sems[b]).wait()
      j = u // cpw
      t0 = wid * tpw + (u % cpw) * 128
      pltpu.sync_copy(bufs[b], out_hbm.at[j, pl.ds(t0, 128)])

    fire(0, 0)

    def step(t, carry):
      u = 2 * t
      fire(u + 1, 1)
      proc(u, 0)
      fire(u + 2, 0)
      proc(u + 1, 1)
      return carry

    lax.fori_loop(0, units // 2 - 1, step, 0)
    fire(units - 1, 1)
    proc(units - 2, 0)
    proc(units - 1, 1)

  return sc_gather


def _tc_unpack_body(x_ref, tok_ref, out_ref):
  eye = _eye(128)
  for c in range(UNPACK_T // 128):
    x = x_ref[0, pl.ds(c * 128, 128)]
    tok = tok_ref[0, pl.ds(c, 1)]
    hbit = jnp.where(tok >= TAIL_START, tok & 1, (tok >> 8) & 1)
    half = hbit == 1
    xt = lax.dot_general(x, eye, (((0,), (0,)), ((), ())))
    z = jnp.where(half, xt[EMB:], xt[:EMB])
    for rt in range(8):
      out_ref[0, rt, c] = z[rt * 8:(rt + 1) * 8]


def _tc_unpack_first_body(x_ref, tok_ref, out_ref):
  _tc_unpack_body(x_ref, tok_ref, out_ref)


def _tc_unpack_rest_body(d_ref, x_ref, tok_ref, out_ref):
  del d_ref
  _tc_unpack_body(x_ref, tok_ref, out_ref)


def _tc_unpack_part(x3, tok_t_part, ct0, d5=None):
  t_rows = x3.shape[1]
  nt = t_rows // UNPACK_T
  nc = UNPACK_T // 128
  tok4 = tok_t_part.reshape(J_COLS * nt, nc, 128)
  out_shape = jax.ShapeDtypeStruct(
      (J_COLS, 8, T_ROWS // 128, 8, 128), jnp.float32
  )
  x_spec = pl.BlockSpec((1, UNPACK_T, 128), lambda j, t: (j, t, 0))
  tok_spec = pl.BlockSpec((1, nc, 128), lambda j, t: (j * nt + t, 0, 0))
  out_spec = pl.BlockSpec(
      (1, 8, nc, 8, 128), lambda j, t: (j, 0, ct0 // nc + t, 0, 0)
  )
  if d5 is None:
    return pl.pallas_call(
        _tc_unpack_first_body,
        grid=(J_COLS, nt),
        in_specs=[x_spec, tok_spec],
        out_specs=out_spec,
        out_shape=out_shape,
    )(x3, tok4)
  return pl.pallas_call(
      _tc_unpack_rest_body,
      grid=(J_COLS, nt),
      in_specs=[
          pl.BlockSpec(memory_space=pl.ANY),
          x_spec,
          tok_spec,
      ],
      out_specs=out_spec,
      out_shape=out_shape,
      input_output_aliases={0: 0},
  )(d5, x3, tok4)


N_PARTS = 2


def kernel(token_ids, weight):
  tok32 = token_ids.astype(jnp.int32)
  tok_t = tok32.T
  wp_main = _tc_pack(weight.T)
  tail = weight[TAIL_START:].reshape(
      (N_VOCAB - TAIL_START) // 2, 2 * EMB
  )
  wp = lax.dynamic_update_slice(wp_main, tail, (TAIL_P0, 0))

  t_part = T_ROWS // N_PARTS
  gather = _make_sc_gather(t_part // NW)
  d5 = None
  for p in range(N_PARTS):
    tok_slice = lax.slice_in_dim(tok_t, p * t_part, (p + 1) * t_part, axis=1)
    tok3 = tok_slice.reshape(J_COLS, NW, t_part // NW)
    x3 = gather(tok3, wp)
    d5 = _tc_unpack_part(x3, tok_slice, p * (t_part // 128), d5)

  e = d5.transpose(0, 1, 3, 2, 4).reshape(J_COLS, EMB, T_ROWS)
  return e.transpose(2, 0, 1)


# speedup vs baseline: 1.0340x; 1.0340x over previous
import functools

import jax
import jax.numpy as jnp
from jax import lax
from jax.experimental import pallas as pl
from jax.experimental.pallas import tpu as pltpu
from jax.experimental.pallas import tpu_sc as plsc

NUM_CORES = 2
NUM_SUBCORES = 16
NW = NUM_CORES * NUM_SUBCORES

T_ROWS = 16384
J_COLS = 26
EMB = 64
N_VOCAB = 1_000_000
N_PAIRS = N_VOCAB // 2

TPW = T_ROWS // NW
CPW = TPW // 128
UNITS = J_COLS * CPW

PACK_COLS = 512
UNPACK_T = 8192
TAIL_START = (N_VOCAB // PACK_COLS) * PACK_COLS
TAIL_P0 = TAIL_START // 2


def _eye(n):
  return (
      lax.broadcasted_iota(jnp.int32, (n, n), 0)
      == lax.broadcasted_iota(jnp.int32, (n, n), 1)
  ).astype(jnp.float32)


PACK_SUB = 63
PACK_BLK = PACK_COLS * PACK_SUB


def _tc_pack_body(wt_ref, out_ref):
  x = wt_ref[...]
  z = lax.dot_general(x, _eye(EMB), (((0,), (0,)), ((), ())))
  for s in range(PACK_SUB):
    out_ref[pl.ds(s * 256, 256), 0:EMB] = z[s * 512:s * 512 + 256]
    out_ref[pl.ds(s * 256, 256), EMB:128] = z[s * 512 + 256:s * 512 + 512]


def _tc_pack(wt):
  grid = N_VOCAB // PACK_BLK
  return pl.pallas_call(
      _tc_pack_body,
      grid=(grid,),
      in_specs=[pl.BlockSpec((EMB, PACK_BLK), lambda k: (0, k))],
      out_specs=pl.BlockSpec((PACK_BLK // 2, 128), lambda k: (k, 0)),
      out_shape=jax.ShapeDtypeStruct((N_PAIRS, 128), jnp.float32),
  )(wt)


def _make_sc_gather(tpw):
  t_rows = tpw * NW
  cpw = tpw // 128
  units = J_COLS * cpw

  @functools.partial(
      pl.kernel,
      out_type=jax.ShapeDtypeStruct((J_COLS, t_rows, 128), jnp.float32),
      mesh=plsc.VectorSubcoreMesh(
          core_axis_name="c",
          subcore_axis_name="s",
          num_cores=NUM_CORES,
          num_subcores=NUM_SUBCORES,
      ),
      scratch_types=[
          pltpu.VMEM((J_COLS, tpw), jnp.int32),
          pltpu.VMEM((J_COLS * tpw,), jnp.int32),
          pltpu.VMEM((128, 128), jnp.float32),
          pltpu.VMEM((128, 128), jnp.float32),
          pltpu.SemaphoreType.DMA,
          pltpu.SemaphoreType.DMA,
      ],
      compiler_params=pltpu.CompilerParams(use_tc_tiling_on_sc=True),
  )
  def sc_gather(tok_hbm, w_hbm, out_hbm, tok_v, idxp, pbuf0, pbuf1,
                sem0, sem1):
    c = lax.axis_index("c")
    s = lax.axis_index("s")
    wid = s * NUM_CORES + c
    sems = (sem0, sem1)
    bufs = (pbuf0, pbuf1)

    pltpu.sync_copy(tok_hbm.at[:, wid], tok_v)

    for j in range(J_COLS):
      def build(g, carry, j=j):
        v = tok_v[j, pl.ds(g * 16, 16)]
        p = lax.shift_left(
            lax.shift_right_logical(v, 9), 8
        ) + lax.bitwise_and(v, 255)
        p_tail = TAIL_P0 + lax.shift_right_logical(v - TAIL_START, 1)
        p = lax.select(v >= TAIL_START, p_tail, p)
        idxp[pl.ds(j * tpw + g * 16, 16)] = p
        return carry
      lax.fori_loop(0, tpw // 16, build, 0)

    def fire(u, b):
      pltpu.async_copy(
          w_hbm.at[idxp.at[pl.ds(u * 128, 128)]], bufs[b], sems[b])

    def proc(u, b):
      pltpu.make_async_copy(
          w_hbm.at[idxp.at[pl.ds(u * 128, 128)]], bufs[b], sems[b]).wait()
      j = u // cpw
      t0 = wid * tpw + (u % cpw) * 128
      pltpu.sync_copy(bufs[b], out_hbm.at[j, pl.ds(t0, 128)])

    fire(0, 0)

    def step(t, carry):
      u = 2 * t
      fire(u + 1, 1)
      proc(u, 0)
      fire(u + 2, 0)
      proc(u + 1, 1)
      return carry

    lax.fori_loop(0, units // 2 - 1, step, 0)
    fire(units - 1, 1)
    proc(units - 2, 0)
    proc(units - 1, 1)

  return sc_gather


def _tc_unpack_body(x_ref, tok_ref, out_ref):
  eye = _eye(128)
  for c in range(UNPACK_T // 128):
    x = x_ref[0, pl.ds(c * 128, 128)]
    tok = tok_ref[0, pl.ds(c, 1)]
    hbit = jnp.where(tok >= TAIL_START, tok & 1, (tok >> 8) & 1)
    half = hbit == 1
    xt = lax.dot_general(x, eye, (((0,), (0,)), ((), ())))
    z = jnp.where(half, xt[EMB:], xt[:EMB])
    for rt in range(8):
      out_ref[0, rt, c] = z[rt * 8:(rt + 1) * 8]


def _tc_unpack_first_body(x_ref, tok_ref, out_ref):
  _tc_unpack_body(x_ref, tok_ref, out_ref)


def _tc_unpack_rest_body(d_ref, x_ref, tok_ref, out_ref):
  del d_ref
  _tc_unpack_body(x_ref, tok_ref, out_ref)


def _tc_unpack_part(x3, tok_t_part, ct0, d5=None):
  t_rows = x3.shape[1]
  nt = t_rows // UNPACK_T
  nc = UNPACK_T // 128
  tok4 = tok_t_part.reshape(J_COLS * nt, nc, 128)
  out_shape = jax.ShapeDtypeStruct(
      (J_COLS, 8, T_ROWS // 128, 8, 128), jnp.float32
  )
  x_spec = pl.BlockSpec((1, UNPACK_T, 128), lambda j, t: (j, t, 0))
  tok_spec = pl.BlockSpec((1, nc, 128), lambda j, t: (j * nt + t, 0, 0))
  out_spec = pl.BlockSpec(
      (1, 8, nc, 8, 128), lambda j, t: (j, 0, ct0 // nc + t, 0, 0)
  )
  if d5 is None:
    return pl.pallas_call(
        _tc_unpack_first_body,
        grid=(J_COLS, nt),
        in_specs=[x_spec, tok_spec],
        out_specs=out_spec,
        out_shape=out_shape,
    )(x3, tok4)
  return pl.pallas_call(
      _tc_unpack_rest_body,
      grid=(J_COLS, nt),
      in_specs=[
          pl.BlockSpec(memory_space=pl.ANY),
          x_spec,
          tok_spec,
      ],
      out_specs=out_spec,
      out_shape=out_shape,
      input_output_aliases={0: 0},
  )(d5, x3, tok4)


N_PARTS = 2


def kernel(token_ids, weight):
  tok32 = token_ids.astype(jnp.int32)
  tok_t = tok32.T
  wp_main = _tc_pack(weight.T)
  tail = weight[TAIL_START:].reshape(
      (N_VOCAB - TAIL_START) // 2, 2 * EMB
  )
  wp = lax.dynamic_update_slice(wp_main, tail, (TAIL_P0, 0))

  t_part = T_ROWS // N_PARTS
  gather = _make_sc_gather(t_part // NW)
  d5 = None
  for p in range(N_PARTS):
    tok_slice = lax.slice_in_dim(tok_t, p * t_part, (p + 1) * t_part, axis=1)
    tok3 = tok_slice.reshape(J_COLS, NW, t_part // NW)
    x3 = gather(tok3, wp)
    d5 = _tc_unpack_part(x3, tok_slice, p * (t_part // 128), d5)

  e = d5.transpose(0, 1, 3, 2, 4).reshape(J_COLS, EMB, T_ROWS)
  return e.transpose(2, 0, 1)
